# baseline (speedup 1.0000x reference)
import functools

import jax
import jax.numpy as jnp
import numpy as np
from jax import lax
from jax.experimental import pallas as pl
from jax.experimental.pallas import tpu as pltpu
from jax.experimental.pallas import tpu_sc as plsc

_B = 16384
_D = 64
_NC = 2
_NS = 16
_NW = _NC * _NS
_W = 1000001
_CPT = 31744
_SW = 512
_FULL_END = 999936
_TAIL = _W - _FULL_END
_STG = 16
_RING = 12
_CAP = _B + 64
_RPW = _B
_DUMMY = _B


def _sweep_one_table(wid, tab_hbm, idx_hbm, out_hbm,
                     idx_v, wl_c, act_c, slab_v, tail_v,
                     stg_r, stg_p, sem_slab, sem_out):
    lane = lax.iota(jnp.int32, 16)
    base = wid * _CPT
    n_slabs = (jnp.minimum(_FULL_END, base + _CPT) - base) // _SW

    def quarter(h, n):
        pltpu.sync_copy(idx_hbm.at[pl.ds(h * 4096, 4096)], idx_v)

        def scan(g, n):
            vec = idx_v[pl.ds(g * 16, 16)]
            mask = (vec >= base) & (vec < base + _CPT)
            pos = lane + (h * 4096 + g * 16)
            packed = (vec - base) * 16384 + pos
            pref = plsc.cumsum(mask.astype(jnp.int32)) - 1
            plsc.store_scatter(wl_c, [n + pref], packed, mask=mask)
            return n + plsc.all_reduce_population_count(mask)[0]

        return lax.fori_loop(0, 256, scan, n)

    n = jnp.int32(0)
    for h in range(4):
        n = quarter(h, n)
    n_groups = (n + 15) // 16

    def make_entry(src, col0):
        def entry(k, carry):
            sl, wr, fc = carry
            full = sl == _STG

            @pl.when(full)
            def _flush():
                sb = fc % _RING
                pltpu.async_copy(stg_r.at[sb], out_hbm.at[stg_p.at[sb]],
                                 sem_out)

                @pl.when(fc >= _RING - 1)
                def _():
                    pltpu.make_async_copy(
                        out_hbm.at[pl.ds(0, _STG)],
                        stg_r.at[0], sem_out).wait()

            wr = jnp.where(full, wr + _STG, wr)
            fc = jnp.where(full, fc + 1, fc)
            sl = jnp.where(full, 0, sl)

            v = act_c[pl.ds(k, 16)][0]
            c_loc = v // 16384 - col0
            p = v & 16383
            sb = fc % _RING
            for q in range(4):
                v = plsc.load_gather(src, [lane + 16 * q, lane * 0 + c_loc])
                stg_r[sb, sl, pl.ds(16 * q, 16)] = v
            plsc.store_scatter(stg_p.at[sb], [lane * 0 + sl], lane * 0 + p,
                               mask=lane == 0)
            return sl + 1, wr, fc

        return entry

    def window_pass(lo_loc, hi_loc, src, carry):
        def group(wg, carry):
            packed = wl_c[pl.ds(wg * 16, 16)]
            cols = packed // 16384
            valid = (lane + wg * 16) < n
            mask = valid & (cols >= lo_loc) & (cols < hi_loc)
            pref = plsc.cumsum(mask.astype(jnp.int32)) - 1
            plsc.store_scatter(act_c, [pref], packed, mask=mask)
            m = plsc.all_reduce_population_count(mask)[0]
            return lax.fori_loop(0, m, make_entry(src, lo_loc), carry)

        return lax.fori_loop(0, n_groups, group, carry)

    def fetch(col0, rb):
        pltpu.async_copy(tab_hbm.at[:, pl.ds(col0, _SW)], slab_v.at[rb],
                         sem_slab)

    @pl.when(n_slabs > 0)
    def _():
        fetch(pl.multiple_of(base, _SW), 0)

    def slab_step(s, carry):
        @pl.when(s + 1 < n_slabs)
        def _():
            fetch(pl.multiple_of(base + (s + 1) * _SW, _SW), (s + 1) & 1)

        pltpu.make_async_copy(tab_hbm.at[:, pl.ds(0, _SW)],
                              slab_v.at[s & 1], sem_slab).wait()

        col0 = s * _SW
        return window_pass(col0, col0 + _SW, slab_v.at[s & 1], carry)

    carry = lax.fori_loop(0, n_slabs, slab_step,
                          (jnp.int32(0), jnp.int32(0), jnp.int32(0)))

    @pl.when(wid == _NW - 1)
    def _():
        pltpu.sync_copy(tab_hbm.at[:, pl.ds(_FULL_END, _TAIL)], tail_v)

    carry = window_pass(_FULL_END - base, _W - base, tail_v, carry)
    sl, wr, fc = carry

    @pl.when(sl > 0)
    def _():
        sb = fc % _RING
        for j in range(_STG // 16):
            seg = lane + 16 * j
            plsc.store_scatter(stg_p.at[sb], [seg], lane * 0 + _DUMMY,
                               mask=seg >= sl)
        pltpu.async_copy(stg_r.at[sb], out_hbm.at[stg_p.at[sb]], sem_out)

    fc_total = fc + jnp.where(sl > 0, 1, 0)

    def drain(i, c):
        pltpu.make_async_copy(out_hbm.at[pl.ds(0, _STG)],
                              stg_r.at[0], sem_out).wait()
        return c

    waited = jnp.maximum(fc - (_RING - 1), 0)
    lax.fori_loop(0, fc_total - waited, drain, jnp.int32(0))


def _sweep_body(uidx, iidx, utab, itab, uout, iout,
                idx_v, wl_c, act_c, slab_v, tail_v,
                stg_r, stg_p, sem_slab, sem_out):
    wid = lax.axis_index("s") * _NC + lax.axis_index("c")
    _sweep_one_table(wid, utab, uidx, uout,
                     idx_v, wl_c, act_c, slab_v, tail_v,
                     stg_r, stg_p, sem_slab, sem_out)
    _sweep_one_table(wid, itab, iidx, iout,
                     idx_v, wl_c, act_c, slab_v, tail_v,
                     stg_r, stg_p, sem_slab, sem_out)


def _sc_sweep(user_idx, item_idx, utab_t, itab_t):
    mesh = plsc.VectorSubcoreMesh(core_axis_name="c", subcore_axis_name="s")
    f = pl.kernel(
        _sweep_body,
        out_type=(
            jax.ShapeDtypeStruct((_B + _STG, 2 * _D), jnp.float32),
            jax.ShapeDtypeStruct((_B + _STG, 2 * _D), jnp.float32),
        ),
        mesh=mesh,
        scratch_types=[
            pltpu.VMEM((4096,), jnp.int32),
            pltpu.VMEM((_CAP,), jnp.int32),
            pltpu.VMEM((48,), jnp.int32),
            pltpu.VMEM((2, _D, _SW), jnp.float32),
            pltpu.VMEM((_D, _TAIL), jnp.float32),
            pltpu.VMEM((_RING, _STG, 2 * _D), jnp.float32),
            pltpu.VMEM((_RING, _STG), jnp.int32),
            pltpu.SemaphoreType.DMA,
            pltpu.SemaphoreType.DMA,
        ],
        compiler_params=pltpu.CompilerParams(use_tc_tiling_on_sc=True,
                                             needs_layout_passes=False),
    )
    return f(user_idx, item_idx, utab_t, itab_t)


def _mlp_body(u_ref, i_ref, w1u_ref, w1i_ref, b1_ref, w2_ref, b2_ref,
              w3_ref, b3_ref, o_ref):
    h = jnp.dot(u_ref[:, :_D], w1u_ref[...],
                preferred_element_type=jnp.float32)
    h = h + jnp.dot(i_ref[:, :_D], w1i_ref[...],
                    preferred_element_type=jnp.float32)
    h = jnp.maximum(h + b1_ref[...], 0.0)
    h2 = jnp.dot(h, w2_ref[...], preferred_element_type=jnp.float32)
    h2 = jnp.maximum(h2 + b2_ref[...], 0.0)
    logit = jnp.dot(h2, w3_ref[...], preferred_element_type=jnp.float32)
    logit = logit + b3_ref[...]
    o_ref[...] = 1.0 / (1.0 + jnp.exp(-logit))


_BM = 2048


def _tc_mlp(u_emb, i_emb, W1, b1, W2, b2, W3, b3):
    full = lambda shape: pl.BlockSpec(shape, lambda ib: (0, 0))
    return pl.pallas_call(
        _mlp_body,
        grid=(_B // _BM,),
        in_specs=[
            pl.BlockSpec((_BM, 2 * _D), lambda ib: (ib, 0)),
            pl.BlockSpec((_BM, 2 * _D), lambda ib: (ib, 0)),
            full((_D, 128)),
            full((_D, 128)),
            full((1, 128)),
            full((128, _D)),
            full((1, _D)),
            full((_D, 1)),
            full((1, 1)),
        ],
        out_specs=pl.BlockSpec((_BM, 1), lambda ib: (ib, 0)),
        out_shape=jax.ShapeDtypeStruct((_B, 1), jnp.float32),
    )(u_emb, i_emb, W1[:_D], W1[_D:], b1.reshape(1, 128),
      W2, b2.reshape(1, _D), W3, b3.reshape(1, 1))


@jax.jit
def kernel(user_input, item_input, user_table, item_table,
           W1, b1, W2, b2, W3, b3):
    u_full, i_full = _sc_sweep(user_input.astype(jnp.int32),
                               item_input.astype(jnp.int32),
                               user_table.T, item_table.T)
    return _tc_mlp(u_full, i_full, W1, b1, W2, b2, W3, b3)


# speedup vs baseline: 1.0096x; 1.0096x over previous
import functools

import jax
import jax.numpy as jnp
import numpy as np
from jax import lax
from jax.experimental import pallas as pl
from jax.experimental.pallas import tpu as pltpu
from jax.experimental.pallas import tpu_sc as plsc

_B = 16384
_D = 64
_NC = 2
_NS = 16
_NW = _NC * _NS
_W = 1000001
_CPT = 31744
_SW = 512
_FULL_END = 999936
_TAIL = _W - _FULL_END
_STG = 24
_RING = 8
_CAP = _B + 64
_RPW = _B
_DUMMY = _B


def _sweep_one_table(wid, tab_hbm, idx_hbm, out_hbm,
                     idx_v, wl_c, act_c, slab_v, tail_v,
                     stg_r, stg_p, sem_slab, sem_out):
    lane = lax.iota(jnp.int32, 16)
    base = wid * _CPT
    n_slabs = (jnp.minimum(_FULL_END, base + _CPT) - base) // _SW

    def quarter(h, n):
        pltpu.sync_copy(idx_hbm.at[pl.ds(h * 4096, 4096)], idx_v)

        def scan(g, n):
            vec = idx_v[pl.ds(g * 16, 16)]
            mask = (vec >= base) & (vec < base + _CPT)
            pos = lane + (h * 4096 + g * 16)
            packed = (vec - base) * 16384 + pos
            pref = plsc.cumsum(mask.astype(jnp.int32)) - 1
            plsc.store_scatter(wl_c, [n + pref], packed, mask=mask)
            return n + plsc.all_reduce_population_count(mask)[0]

        return lax.fori_loop(0, 256, scan, n)

    n = jnp.int32(0)
    for h in range(4):
        n = quarter(h, n)
    n_groups = (n + 15) // 16

    def make_entry(src, col0):
        def entry(k, carry):
            sl, wr, fc = carry
            full = sl == _STG

            @pl.when(full)
            def _flush():
                sb = fc % _RING
                pltpu.async_copy(stg_r.at[sb], out_hbm.at[stg_p.at[sb]],
                                 sem_out)

                @pl.when(fc >= _RING - 1)
                def _():
                    pltpu.make_async_copy(
                        out_hbm.at[pl.ds(0, _STG)],
                        stg_r.at[0], sem_out).wait()

            wr = jnp.where(full, wr + _STG, wr)
            fc = jnp.where(full, fc + 1, fc)
            sl = jnp.where(full, 0, sl)

            v = act_c[pl.ds(k, 16)][0]
            c_loc = v // 16384 - col0
            p = v & 16383
            sb = fc % _RING
            for q in range(4):
                v = plsc.load_gather(src, [lane + 16 * q, lane * 0 + c_loc])
                stg_r[sb, sl, pl.ds(16 * q, 16)] = v
            plsc.store_scatter(stg_p.at[sb], [lane * 0 + sl], lane * 0 + p,
                               mask=lane == 0)
            return sl + 1, wr, fc

        return entry

    def window_pass(lo_loc, hi_loc, src, carry):
        def group(wg, carry):
            packed = wl_c[pl.ds(wg * 16, 16)]
            cols = packed // 16384
            valid = (lane + wg * 16) < n
            mask = valid & (cols >= lo_loc) & (cols < hi_loc)
            pref = plsc.cumsum(mask.astype(jnp.int32)) - 1
            plsc.store_scatter(act_c, [pref], packed, mask=mask)
            m = plsc.all_reduce_population_count(mask)[0]
            return lax.fori_loop(0, m, make_entry(src, lo_loc), carry)

        return lax.fori_loop(0, n_groups, group, carry)

    def fetch(col0, rb):
        pltpu.async_copy(tab_hbm.at[:, pl.ds(col0, _SW)], slab_v.at[rb],
                         sem_slab)

    @pl.when(n_slabs > 0)
    def _():
        fetch(pl.multiple_of(base, _SW), 0)

    def slab_step(s, carry):
        @pl.when(s + 1 < n_slabs)
        def _():
            fetch(pl.multiple_of(base + (s + 1) * _SW, _SW), (s + 1) & 1)

        pltpu.make_async_copy(tab_hbm.at[:, pl.ds(0, _SW)],
                              slab_v.at[s & 1], sem_slab).wait()

        col0 = s * _SW
        return window_pass(col0, col0 + _SW, slab_v.at[s & 1], carry)

    carry = lax.fori_loop(0, n_slabs, slab_step,
                          (jnp.int32(0), jnp.int32(0), jnp.int32(0)))

    @pl.when(wid == _NW - 1)
    def _():
        pltpu.sync_copy(tab_hbm.at[:, pl.ds(_FULL_END, _TAIL)], tail_v)

    carry = window_pass(_FULL_END - base, _W - base, tail_v, carry)
    sl, wr, fc = carry

    @pl.when(sl > 0)
    def _():
        sb = fc % _RING
        for j in range(_STG // 16):
            seg = lane + 16 * j
            plsc.store_scatter(stg_p.at[sb], [seg], lane * 0 + _DUMMY,
                               mask=seg >= sl)
        pltpu.async_copy(stg_r.at[sb], out_hbm.at[stg_p.at[sb]], sem_out)

    fc_total = fc + jnp.where(sl > 0, 1, 0)

    def drain(i, c):
        pltpu.make_async_copy(out_hbm.at[pl.ds(0, _STG)],
                              stg_r.at[0], sem_out).wait()
        return c

    waited = jnp.maximum(fc - (_RING - 1), 0)
    lax.fori_loop(0, fc_total - waited, drain, jnp.int32(0))


def _sweep_body(uidx, iidx, utab, itab, uout, iout,
                idx_v, wl_c, act_c, slab_v, tail_v,
                stg_r, stg_p, sem_slab, sem_out):
    wid = lax.axis_index("s") * _NC + lax.axis_index("c")
    _sweep_one_table(wid, utab, uidx, uout,
                     idx_v, wl_c, act_c, slab_v, tail_v,
                     stg_r, stg_p, sem_slab, sem_out)
    _sweep_one_table(wid, itab, iidx, iout,
                     idx_v, wl_c, act_c, slab_v, tail_v,
                     stg_r, stg_p, sem_slab, sem_out)


def _sc_sweep(user_idx, item_idx, utab_t, itab_t):
    mesh = plsc.VectorSubcoreMesh(core_axis_name="c", subcore_axis_name="s")
    f = pl.kernel(
        _sweep_body,
        out_type=(
            jax.ShapeDtypeStruct((_B + _STG, 2 * _D), jnp.float32),
            jax.ShapeDtypeStruct((_B + _STG, 2 * _D), jnp.float32),
        ),
        mesh=mesh,
        scratch_types=[
            pltpu.VMEM((4096,), jnp.int32),
            pltpu.VMEM((_CAP,), jnp.int32),
            pltpu.VMEM((48,), jnp.int32),
            pltpu.VMEM((2, _D, _SW), jnp.float32),
            pltpu.VMEM((_D, _TAIL), jnp.float32),
            pltpu.VMEM((_RING, _STG, 2 * _D), jnp.float32),
            pltpu.VMEM((_RING, _STG), jnp.int32),
            pltpu.SemaphoreType.DMA,
            pltpu.SemaphoreType.DMA,
        ],
        compiler_params=pltpu.CompilerParams(use_tc_tiling_on_sc=True,
                                             needs_layout_passes=False),
    )
    return f(user_idx, item_idx, utab_t, itab_t)


def _mlp_body(u_ref, i_ref, w1u_ref, w1i_ref, b1_ref, w2_ref, b2_ref,
              w3_ref, b3_ref, o_ref):
    h = jnp.dot(u_ref[:, :_D], w1u_ref[...],
                preferred_element_type=jnp.float32)
    h = h + jnp.dot(i_ref[:, :_D], w1i_ref[...],
                    preferred_element_type=jnp.float32)
    h = jnp.maximum(h + b1_ref[...], 0.0)
    h2 = jnp.dot(h, w2_ref[...], preferred_element_type=jnp.float32)
    h2 = jnp.maximum(h2 + b2_ref[...], 0.0)
    logit = jnp.dot(h2, w3_ref[...], preferred_element_type=jnp.float32)
    logit = logit + b3_ref[...]
    o_ref[...] = 1.0 / (1.0 + jnp.exp(-logit))


_BM = 2048


def _tc_mlp(u_emb, i_emb, W1, b1, W2, b2, W3, b3):
    full = lambda shape: pl.BlockSpec(shape, lambda ib: (0, 0))
    return pl.pallas_call(
        _mlp_body,
        grid=(_B // _BM,),
        in_specs=[
            pl.BlockSpec((_BM, 2 * _D), lambda ib: (ib, 0)),
            pl.BlockSpec((_BM, 2 * _D), lambda ib: (ib, 0)),
            full((_D, 128)),
            full((_D, 128)),
            full((1, 128)),
            full((128, _D)),
            full((1, _D)),
            full((_D, 1)),
            full((1, 1)),
        ],
        out_specs=pl.BlockSpec((_BM, 1), lambda ib: (ib, 0)),
        out_shape=jax.ShapeDtypeStruct((_B, 1), jnp.float32),
    )(u_emb, i_emb, W1[:_D], W1[_D:], b1.reshape(1, 128),
      W2, b2.reshape(1, _D), W3, b3.reshape(1, 1))


@jax.jit
def kernel(user_input, item_input, user_table, item_table,
           W1, b1, W2, b2, W3, b3):
    u_full, i_full = _sc_sweep(user_input.astype(jnp.int32),
                               item_input.astype(jnp.int32),
                               user_table.T, item_table.T)
    return _tc_mlp(u_full, i_full, W1, b1, W2, b2, W3, b3)


# speedup vs baseline: 1.0099x; 1.0002x over previous
import jax
import jax.numpy as jnp
from jax import lax
from jax.experimental import pallas as pl
from jax.experimental.pallas import tpu as pltpu
from jax.experimental.pallas import tpu_sc as plsc

_B = 16384
_D = 64
_NC = 2
_NS = 16
_NW = _NC * _NS
_W = 1000001
_CPT = 31744
_SW = 512
_FULL_END = 999936
_TAIL = _W - _FULL_END
_STG = 24
_RING = 8
_CAP = _B + 64
_DUMMY = _B


def _sweep_one_table(wid, tab_hbm, idx_hbm, out_hbm,
                     idx_v, wl_c, act_c, slab_v, tail_v,
                     stg_r, stg_p, sem_slab, sem_out):
    lane = lax.iota(jnp.int32, 16)
    base = wid * _CPT
    n_slabs = (jnp.minimum(_FULL_END, base + _CPT) - base) // _SW

    def quarter(h, n):
        pltpu.sync_copy(idx_hbm.at[pl.ds(h * 4096, 4096)], idx_v)

        def scan(g, n):
            vec = idx_v[pl.ds(g * 16, 16)]
            mask = (vec >= base) & (vec < base + _CPT)
            pos = lane + (h * 4096 + g * 16)
            packed = (vec - base) * 16384 + pos
            pref = plsc.cumsum(mask.astype(jnp.int32)) - 1
            plsc.store_scatter(wl_c, [n + pref], packed, mask=mask)
            return n + plsc.all_reduce_population_count(mask)[0]

        return lax.fori_loop(0, 256, scan, n)

    n = jnp.int32(0)
    for h in range(4):
        n = quarter(h, n)
    n_groups = (n + 15) // 16

    def make_entry(src, col0):
        def entry(k, carry):
            sl, wr, fc = carry
            full = sl == _STG

            @pl.when(full)
            def _flush():
                sb = fc % _RING
                pltpu.async_copy(stg_r.at[sb], out_hbm.at[stg_p.at[sb]],
                                 sem_out)

                @pl.when(fc >= _RING - 1)
                def _():
                    pltpu.make_async_copy(
                        out_hbm.at[pl.ds(0, _STG)],
                        stg_r.at[0], sem_out).wait()

            wr = jnp.where(full, wr + _STG, wr)
            fc = jnp.where(full, fc + 1, fc)
            sl = jnp.where(full, 0, sl)

            v = act_c[pl.ds(k, 16)][0]
            c_loc = v // 16384 - col0
            p = v & 16383
            sb = fc % _RING
            for q in range(4):
                v = plsc.load_gather(src, [lane + 16 * q, lane * 0 + c_loc])
                stg_r[sb, sl, pl.ds(16 * q, 16)] = v
            plsc.store_scatter(stg_p.at[sb], [lane * 0 + sl], lane * 0 + p,
                               mask=lane == 0)
            return sl + 1, wr, fc

        return entry

    def window_pass(lo_loc, hi_loc, src, carry):
        def group(wg, carry):
            packed = wl_c[pl.ds(wg * 16, 16)]
            cols = packed // 16384
            valid = (lane + wg * 16) < n
            mask = valid & (cols >= lo_loc) & (cols < hi_loc)
            pref = plsc.cumsum(mask.astype(jnp.int32)) - 1
            plsc.store_scatter(act_c, [pref], packed, mask=mask)
            m = plsc.all_reduce_population_count(mask)[0]
            return lax.fori_loop(0, m, make_entry(src, lo_loc), carry)

        return lax.fori_loop(0, n_groups, group, carry)

    def fetch(col0, rb):
        pltpu.async_copy(tab_hbm.at[:, pl.ds(col0, _SW)], slab_v.at[rb],
                         sem_slab)

    @pl.when(n_slabs > 0)
    def _():
        fetch(pl.multiple_of(base, _SW), 0)

    def slab_step(s, carry):
        @pl.when(s + 1 < n_slabs)
        def _():
            fetch(pl.multiple_of(base + (s + 1) * _SW, _SW), (s + 1) & 1)

        pltpu.make_async_copy(tab_hbm.at[:, pl.ds(0, _SW)],
                              slab_v.at[s & 1], sem_slab).wait()

        col0 = s * _SW
        return window_pass(col0, col0 + _SW, slab_v.at[s & 1], carry)

    carry = lax.fori_loop(0, n_slabs, slab_step,
                          (jnp.int32(0), jnp.int32(0), jnp.int32(0)))

    @pl.when(wid == _NW - 1)
    def _():
        pltpu.sync_copy(tab_hbm.at[:, pl.ds(_FULL_END, _TAIL)], tail_v)

    carry = window_pass(_FULL_END - base, _W - base, tail_v, carry)
    sl, wr, fc = carry

    @pl.when(sl > 0)
    def _():
        sb = fc % _RING
        for j in range(_STG // 16):
            seg = lane + 16 * j
            plsc.store_scatter(stg_p.at[sb], [seg], lane * 0 + _DUMMY,
                               mask=seg >= sl)
        pltpu.async_copy(stg_r.at[sb], out_hbm.at[stg_p.at[sb]], sem_out)

    fc_total = fc + jnp.where(sl > 0, 1, 0)

    def drain(i, c):
        pltpu.make_async_copy(out_hbm.at[pl.ds(0, _STG)],
                              stg_r.at[0], sem_out).wait()
        return c

    waited = jnp.maximum(fc - (_RING - 1), 0)
    lax.fori_loop(0, fc_total - waited, drain, jnp.int32(0))


def _sweep_body(uidx, iidx, utab, itab, uout, iout,
                idx_v, wl_c, act_c, slab_v, tail_v,
                stg_r, stg_p, sem_slab, sem_out):
    wid = lax.axis_index("s") * _NC + lax.axis_index("c")
    _sweep_one_table(wid, utab, uidx, uout,
                     idx_v, wl_c, act_c, slab_v, tail_v,
                     stg_r, stg_p, sem_slab, sem_out)
    _sweep_one_table(wid, itab, iidx, iout,
                     idx_v, wl_c, act_c, slab_v, tail_v,
                     stg_r, stg_p, sem_slab, sem_out)


def _sc_sweep(user_idx, item_idx, utab_t, itab_t):
    mesh = plsc.VectorSubcoreMesh(core_axis_name="c", subcore_axis_name="s")
    f = pl.kernel(
        _sweep_body,
        out_type=(
            jax.ShapeDtypeStruct((_B + _STG, 2 * _D), jnp.float32),
            jax.ShapeDtypeStruct((_B + _STG, 2 * _D), jnp.float32),
        ),
        mesh=mesh,
        scratch_types=[
            pltpu.VMEM((4096,), jnp.int32),
            pltpu.VMEM((_CAP,), jnp.int32),
            pltpu.VMEM((48,), jnp.int32),
            pltpu.VMEM((2, _D, _SW), jnp.float32),
            pltpu.VMEM((_D, _TAIL), jnp.float32),
            pltpu.VMEM((_RING, _STG, 2 * _D), jnp.float32),
            pltpu.VMEM((_RING, _STG), jnp.int32),
            pltpu.SemaphoreType.DMA,
            pltpu.SemaphoreType.DMA,
        ],
        compiler_params=pltpu.CompilerParams(use_tc_tiling_on_sc=True,
                                             needs_layout_passes=False),
    )
    return f(user_idx, item_idx, utab_t, itab_t)


def _mlp_body(u_ref, i_ref, w1u_ref, w1i_ref, b1_ref, w2_ref, b2_ref,
              w3_ref, b3_ref, o_ref):
    h = jnp.dot(u_ref[:, :_D], w1u_ref[...],
                preferred_element_type=jnp.float32)
    h = h + jnp.dot(i_ref[:, :_D], w1i_ref[...],
                    preferred_element_type=jnp.float32)
    h = jnp.maximum(h + b1_ref[...], 0.0)
    h2 = jnp.dot(h, w2_ref[...], preferred_element_type=jnp.float32)
    h2 = jnp.maximum(h2 + b2_ref[...], 0.0)
    logit = jnp.dot(h2, w3_ref[...], preferred_element_type=jnp.float32)
    logit = logit + b3_ref[...]
    o_ref[...] = 1.0 / (1.0 + jnp.exp(-logit))


_BM = 2048


def _tc_mlp(u_emb, i_emb, W1, b1, W2, b2, W3, b3):
    full = lambda shape: pl.BlockSpec(shape, lambda ib: (0, 0))
    return pl.pallas_call(
        _mlp_body,
        grid=(_B // _BM,),
        in_specs=[
            pl.BlockSpec((_BM, 2 * _D), lambda ib: (ib, 0)),
            pl.BlockSpec((_BM, 2 * _D), lambda ib: (ib, 0)),
            full((_D, 128)),
            full((_D, 128)),
            full((1, 128)),
            full((128, _D)),
            full((1, _D)),
            full((_D, 1)),
            full((1, 1)),
        ],
        out_specs=pl.BlockSpec((_BM, 1), lambda ib: (ib, 0)),
        out_shape=jax.ShapeDtypeStruct((_B, 1), jnp.float32),
    )(u_emb, i_emb, W1[:_D], W1[_D:], b1.reshape(1, 128),
      W2, b2.reshape(1, _D), W3, b3.reshape(1, 1))


@jax.jit
def kernel(user_input, item_input, user_table, item_table,
           W1, b1, W2, b2, W3, b3):
    u_full, i_full = _sc_sweep(user_input.astype(jnp.int32),
                               item_input.astype(jnp.int32),
                               user_table.T, item_table.T)
    return _tc_mlp(u_full, i_full, W1, b1, W2, b2, W3, b3)
